# baseline (speedup 1.0000x reference)
import jax
import jax.numpy as jnp
from jax.experimental import pallas as pl
from jax.experimental.pallas import tpu as pltpu
from jax.experimental.pallas import tpu_sc as plsc

_B, _S, _D, _H, _HD, _E, _K, _FF = 2, 2048, 1024, 16, 64, 8, 2, 4096
_T = _B * _S
_CAP = int(_B * _S * 1.25 * _K / _E)
_NSLOT = _E * _CAP
_DROWS = _NSLOT + 256
_TEMP = 0.1
_BF = jnp.bfloat16
_F32 = jnp.float32


def _qkvg_body(x_ref, sc_ref, bi_ref, w_ref, o_ref):
    x = x_ref[...]
    m = jnp.mean(x, axis=1, keepdims=True)
    v = jnp.mean((x - m) ** 2, axis=1, keepdims=True)
    h = (x - m) * jax.lax.rsqrt(v + 1e-5) * sc_ref[...] + bi_ref[...]
    o_ref[...] = jnp.dot(h.astype(_BF), w_ref[...],
                         preferred_element_type=_F32).astype(_BF)


def _qkvg_call(x, scale, bias, wcat):
    return pl.pallas_call(
        _qkvg_body,
        grid=(8, 8),
        in_specs=[
            pl.BlockSpec((512, _D), lambda i, j: (i, 0)),
            pl.BlockSpec((1, _D), lambda i, j: (0, 0)),
            pl.BlockSpec((1, _D), lambda i, j: (0, 0)),
            pl.BlockSpec((_D, 512), lambda i, j: (0, j)),
        ],
        out_specs=pl.BlockSpec((512, 512), lambda i, j: (i, j)),
        out_shape=jax.ShapeDtypeStruct((_T, 4 * _D), _BF),
        compiler_params=pltpu.CompilerParams(
            dimension_semantics=("parallel", "parallel")),
    )(x, scale, bias, wcat)


_BQ = 256


def _rope(x, cos, sin):
    half = _HD // 2
    x1, x2 = x[:, :half], x[:, half:]
    return jnp.concatenate([x1 * cos - x2 * sin, x1 * sin + x2 * cos], axis=1)


def _flash_body(q_ref, k_ref, v_ref, cs_ref, sn_ref, o_ref):
    i = pl.program_id(1)
    cos_q = cs_ref[pl.ds(i * _BQ, _BQ), :]
    sin_q = sn_ref[pl.ds(i * _BQ, _BQ), :]
    q = _rope(q_ref[...].reshape(_BQ, _HD).astype(_F32), cos_q, sin_q) * 0.125
    k = _rope(k_ref[...].reshape(_S, _HD).astype(_F32), cs_ref[...], sn_ref[...])
    s = jax.lax.dot_general(q.astype(_BF), k.astype(_BF),
                            (((1,), (1,)), ((), ())),
                            preferred_element_type=_F32)
    qpos = i * _BQ + jax.lax.broadcasted_iota(jnp.int32, (_BQ, _S), 0)
    kpos = jax.lax.broadcasted_iota(jnp.int32, (_BQ, _S), 1)
    s = jnp.where(qpos >= kpos, s, -1e9)
    m = jnp.max(s, axis=1, keepdims=True)
    p = jnp.exp(s - m)
    l = jnp.sum(p, axis=1, keepdims=True)
    v = v_ref[...].reshape(_S, _HD)
    o = jnp.dot(p.astype(_BF), v, preferred_element_type=_F32) / l
    o_ref[...] = o.astype(_BF).reshape(_BQ, 1, 1, _HD)


def _flash_call(qkvg, cos, sin):
    nq = _S // _BQ
    qkvg4 = qkvg.reshape(_T, 64, 1, _HD)
    attn4 = pl.pallas_call(
        _flash_body,
        grid=(_B * _H, nq),
        in_specs=[
            pl.BlockSpec((_BQ, 1, 1, _HD),
                         lambda bh, i: (bh // _H * nq + i, bh % _H, 0, 0)),
            pl.BlockSpec((_S, 1, 1, _HD),
                         lambda bh, i: (bh // _H, _H + bh % _H, 0, 0)),
            pl.BlockSpec((_S, 1, 1, _HD),
                         lambda bh, i: (bh // _H, 2 * _H + bh % _H, 0, 0)),
            pl.BlockSpec((_S, _HD // 2), lambda bh, i: (0, 0)),
            pl.BlockSpec((_S, _HD // 2), lambda bh, i: (0, 0)),
        ],
        out_specs=pl.BlockSpec((_BQ, 1, 1, _HD),
                               lambda bh, i: (bh // _H * nq + i, bh % _H, 0, 0)),
        out_shape=jax.ShapeDtypeStruct((_T, _H, 1, _HD), _BF),
        compiler_params=pltpu.CompilerParams(
            dimension_semantics=("parallel", "parallel")),
    )(qkvg4, qkvg4, qkvg4, cos, sin)
    return attn4.reshape(_T, _H * _HD)


def _wo_body(a_ref, wo_ref, g_ref, bg_ref, ls_ref, x_ref, o_ref):
    a = jnp.dot(a_ref[...], wo_ref[...], preferred_element_type=_F32)
    gate = jax.nn.sigmoid(g_ref[...].astype(_F32) + bg_ref[...])
    o_ref[...] = x_ref[...] + ls_ref[...] * (gate * a)


def _wo_call(attn, wo, qkvg, bg, ls1, x):
    return pl.pallas_call(
        _wo_body,
        grid=(8, 4),
        in_specs=[
            pl.BlockSpec((512, _D), lambda i, j: (i, 0)),
            pl.BlockSpec((_D, 256), lambda i, j: (0, j)),
            pl.BlockSpec((512, 256), lambda i, j: (i, 12 + j)),
            pl.BlockSpec((1, 256), lambda i, j: (0, j)),
            pl.BlockSpec((1, 256), lambda i, j: (0, j)),
            pl.BlockSpec((512, 256), lambda i, j: (i, j)),
        ],
        out_specs=pl.BlockSpec((512, 256), lambda i, j: (i, j)),
        out_shape=jax.ShapeDtypeStruct((_T, _D), _F32),
        compiler_params=pltpu.CompilerParams(
            dimension_semantics=("parallel", "parallel")),
    )(attn, wo, qkvg, bg, ls1, x)


def _ln2_body(x_ref, sc_ref, bi_ref, wr_ref, h2_ref, tw_ref, ti_ref):
    x = x_ref[...]
    m = jnp.mean(x, axis=1, keepdims=True)
    v = jnp.mean((x - m) ** 2, axis=1, keepdims=True)
    h = (x - m) * jax.lax.rsqrt(v + 1e-5) * sc_ref[...] + bi_ref[...]
    h2_ref[...] = h
    logits = jnp.dot(h, wr_ref[...], preferred_element_type=_F32) / _TEMP
    lane = jax.lax.broadcasted_iota(jnp.int32, logits.shape, 1)
    valid = lane < _E
    mx = jnp.max(jnp.where(valid, logits, -1e30), axis=1, keepdims=True)
    p = jnp.where(valid, jnp.exp(logits - mx), 0.0)
    probs = p / jnp.sum(p, axis=1, keepdims=True)
    v1 = jnp.max(probs, axis=1, keepdims=True)
    i1 = jnp.min(jnp.where(probs == v1, lane, 128), axis=1, keepdims=True)
    p2 = jnp.where(lane == i1, -1.0, probs)
    v2 = jnp.max(p2, axis=1, keepdims=True)
    i2 = jnp.min(jnp.where(p2 == v2, lane, 128), axis=1, keepdims=True)
    tot = v1 + v2
    k8 = jax.lax.broadcasted_iota(jnp.int32, (x.shape[0], 8), 1)
    tw_ref[...] = jnp.where(k8 == 0, v1 / tot, jnp.where(k8 == 1, v2 / tot, 0.0))
    ti_ref[...] = jnp.where(k8 == 0, i1, jnp.where(k8 == 1, i2, 0))


def _ln2_call(x2, scale, bias, wr_pad):
    return pl.pallas_call(
        _ln2_body,
        grid=(16,),
        in_specs=[
            pl.BlockSpec((256, _D), lambda i: (i, 0)),
            pl.BlockSpec((1, _D), lambda i: (0, 0)),
            pl.BlockSpec((1, _D), lambda i: (0, 0)),
            pl.BlockSpec((_D, 128), lambda i: (0, 0)),
        ],
        out_specs=[
            pl.BlockSpec((256, _D), lambda i: (i, 0)),
            pl.BlockSpec((256, 8), lambda i: (i, 0)),
            pl.BlockSpec((256, 8), lambda i: (i, 0)),
        ],
        out_shape=[
            jax.ShapeDtypeStruct((_T, _D), _F32),
            jax.ShapeDtypeStruct((_T, 8), _F32),
            jax.ShapeDtypeStruct((_T, 8), jnp.int32),
        ],
        compiler_params=pltpu.CompilerParams(
            dimension_semantics=("parallel",)),
    )(x2, scale, bias, wr_pad)


_PC = 1024


def _pos_body(fe_ref, ss_ref, sg_ref, kp_ref):
    r = jax.lax.broadcasted_iota(jnp.int32, (_PC, _PC), 0)
    c = jax.lax.broadcasted_iota(jnp.int32, (_PC, _PC), 1)
    tri = (r >= c).astype(_F32)
    lane = jax.lax.broadcasted_iota(jnp.int32, (_PC, 128), 1)

    q4 = jax.lax.broadcasted_iota(jnp.int32, (_PC, 4), 1)

    def chunk(ci, carry):
        e = fe_ref[pl.ds(ci * _PC, _PC), :]
        ohc = (e == lane).astype(_F32)
        cum = jnp.dot(tri, ohc, preferred_element_type=_F32) + carry
        pos = jnp.sum(ohc * cum, axis=1, keepdims=True).astype(jnp.int32) - 1
        keep = pos < _CAP
        slot = e * _CAP + jnp.minimum(pos, _CAP - 1)
        ss_ref[pl.ds(ci * _PC, _PC), :] = jnp.where(keep, slot, _NSLOT) * 4 + q4
        sg_ref[pl.ds(ci * _PC, _PC), :] = jnp.where(keep, slot, 0) * 4 + q4
        kp_ref[pl.ds(ci * _PC, _PC), :] = keep.astype(_F32)
        return carry + jnp.sum(ohc, axis=0, keepdims=True)

    jax.lax.fori_loop(0, (2 * _T) // _PC, chunk, jnp.zeros((1, 128), _F32))


def _pos_call(fe):
    return pl.pallas_call(
        _pos_body,
        in_specs=[pl.BlockSpec((2 * _T, 1), lambda: (0, 0))],
        out_specs=[
            pl.BlockSpec((2 * _T, 4), lambda: (0, 0)),
            pl.BlockSpec((2 * _T, 4), lambda: (0, 0)),
            pl.BlockSpec((2 * _T, 1), lambda: (0, 0)),
        ],
        out_shape=[
            jax.ShapeDtypeStruct((2 * _T, 4), jnp.int32),
            jax.ShapeDtypeStruct((2 * _T, 4), jnp.int32),
            jax.ShapeDtypeStruct((2 * _T, 1), _F32),
        ],
    )(fe)


def _sc_mesh():
    return plsc.VectorSubcoreMesh(core_axis_name="core",
                                  subcore_axis_name="subcore")


def _dispatch_scatter(h2q, slots4):
    nw = (4 * _T) // 128

    @pl.kernel(out_type=jax.ShapeDtypeStruct((4 * _DROWS, _D // 4), _F32),
               mesh=_sc_mesh(), scratch_types=[])
    def k(x_hbm, i_hbm, o_hbm):
        def body(x_vmem, i_vmem):
            pltpu.sync_copy(x_vmem, o_hbm.at[i_vmem.at[0]])

        pltpu.emit_pipeline(
            body,
            grid=(_K, nw),
            in_specs=[
                pl.BlockSpec((128, _D // 4), index_map=lambda kk, i: (i, 0)),
                pl.BlockSpec((1, 128), index_map=lambda kk, i: (0, kk * nw + i)),
            ],
            out_specs=[],
            core_axis_name=("core", "subcore"),
            dimension_semantics=(pltpu.PARALLEL, pltpu.PARALLEL),
        )(x_hbm, i_hbm)

    return k(h2q, slots4)


def _combine_gather(eoutq, slots4):
    nw = (2 * 4 * _T) // 128

    @pl.kernel(out_type=jax.ShapeDtypeStruct((2 * 4 * _T, _D // 4), _F32),
               mesh=_sc_mesh(), scratch_types=[])
    def k(x_hbm, i_hbm, o_hbm):
        def body(i_vmem, o_vmem):
            pltpu.sync_copy(x_hbm.at[i_vmem.at[0]], o_vmem)

        pltpu.emit_pipeline(
            body,
            grid=(nw,),
            in_specs=[pl.BlockSpec((1, 128), index_map=lambda i: (0, i))],
            out_specs=[pl.BlockSpec((128, _D // 4), index_map=lambda i: (i, 0))],
            core_axis_name=("core", "subcore"),
            dimension_semantics=(pltpu.PARALLEL,),
        )(i_hbm, o_hbm)

    return k(eoutq, slots4)


def _ffn_body(d_ref, w1_ref, b1_ref, w2_ref, b2_ref, o_ref):
    h = jnp.dot(d_ref[...].astype(_BF), w1_ref[0],
                preferred_element_type=_F32) + b1_ref[0]
    h = jnp.maximum(h, 0.0).astype(_BF)
    o_ref[...] = jnp.dot(h, w2_ref[0], preferred_element_type=_F32) + b2_ref[0]


def _ffn_call(disp, w1, b1, w2, b2):
    nc = _CAP // 256
    return pl.pallas_call(
        _ffn_body,
        grid=(_E, nc),
        in_specs=[
            pl.BlockSpec((256, _D), lambda e, c: (e * nc + c, 0)),
            pl.BlockSpec((1, _D, _FF), lambda e, c: (e, 0, 0)),
            pl.BlockSpec((1, 1, _FF), lambda e, c: (e, 0, 0)),
            pl.BlockSpec((1, _FF, _D), lambda e, c: (e, 0, 0)),
            pl.BlockSpec((1, 1, _D), lambda e, c: (e, 0, 0)),
        ],
        out_specs=pl.BlockSpec((256, _D), lambda e, c: (e * nc + c, 0)),
        out_shape=jax.ShapeDtypeStruct((_NSLOT, _D), _F32),
        compiler_params=pltpu.CompilerParams(
            dimension_semantics=("parallel", "parallel")),
    )(disp, w1, b1, w2, b2)


def _comb_body(g0_ref, g1_ref, tw_ref, k0_ref, k1_ref, x_ref, ls_ref, o_ref):
    w0 = tw_ref[...][:, 0:1] * k0_ref[...]
    w1 = tw_ref[...][:, 1:2] * k1_ref[...]
    moe = w0 * g0_ref[...] + w1 * g1_ref[...]
    o_ref[...] = x_ref[...] + ls_ref[...] * moe


def _comb_call(g0, g1, tw, k0, k1, x2, ls2):
    return pl.pallas_call(
        _comb_body,
        grid=(8, 4),
        in_specs=[
            pl.BlockSpec((512, 256), lambda i, j: (i, j)),
            pl.BlockSpec((512, 256), lambda i, j: (i, j)),
            pl.BlockSpec((512, 8), lambda i, j: (i, 0)),
            pl.BlockSpec((512, 1), lambda i, j: (i, 0)),
            pl.BlockSpec((512, 1), lambda i, j: (i, 0)),
            pl.BlockSpec((512, 256), lambda i, j: (i, j)),
            pl.BlockSpec((1, 256), lambda i, j: (0, j)),
        ],
        out_specs=pl.BlockSpec((512, 256), lambda i, j: (i, j)),
        out_shape=jax.ShapeDtypeStruct((_T, _D), _F32),
        compiler_params=pltpu.CompilerParams(
            dimension_semantics=("parallel", "parallel")),
    )(g0, g1, tw, k0, k1, x2, ls2)


def kernel(hidden_states, ln1_scale, ln1_bias, Wq, Wk, Wv, Wo, Wg, bg, ls1,
           ln2_scale, ln2_bias, Wr, w1, b1, w2, b2, ls2):
    x = hidden_states.reshape(_T, _D)
    wcat = jnp.concatenate([Wq, Wk, Wv, Wg], axis=1).astype(_BF)
    qkvg = _qkvg_call(x, ln1_scale.reshape(1, _D), ln1_bias.reshape(1, _D), wcat)

    half = _HD // 2
    posn = jnp.arange(_S, dtype=_F32)[:, None]
    freqs = 1.0 / (10000.0 ** (jnp.arange(half, dtype=_F32) / half))
    ang = posn * freqs
    cos, sin = jnp.cos(ang), jnp.sin(ang)

    attn = _flash_call(qkvg, cos, sin)
    x2 = _wo_call(attn, Wo.astype(_BF), qkvg, bg.reshape(1, _D),
                  ls1.reshape(1, _D), x)

    wr_pad = jnp.pad(Wr, ((0, 0), (0, 128 - _E)))
    h2b, tw, ti = _ln2_call(x2, ln2_scale.reshape(1, _D),
                            ln2_bias.reshape(1, _D), wr_pad)

    fe = jnp.concatenate([ti[:, 0], ti[:, 1]]).reshape(2 * _T, 1)
    ss, sg, kp = _pos_call(fe)

    disp4 = _dispatch_scatter(h2b.reshape(4 * _T, _D // 4),
                              ss.reshape(1, 4 * 2 * _T))
    eout = _ffn_call(disp4.reshape(_DROWS, _D), w1.astype(_BF),
                     b1.reshape(_E, 1, _FF), w2.astype(_BF),
                     b2.reshape(_E, 1, _D))
    gathered = _combine_gather(eout.reshape(4 * _NSLOT, _D // 4),
                               sg.reshape(1, 4 * 2 * _T)).reshape(2 * _T, _D)

    xo = _comb_call(gathered[:_T], gathered[_T:], tw, kp[:_T], kp[_T:],
                    x2, ls2.reshape(1, _D))
    return xo.reshape(_B, _S, _D)


# speedup vs baseline: 1.6453x; 1.6453x over previous
import jax
import jax.numpy as jnp
from jax.experimental import pallas as pl
from jax.experimental.pallas import tpu as pltpu
from jax.experimental.pallas import tpu_sc as plsc

_B, _S, _D, _H, _HD, _E, _K, _FF = 2, 2048, 1024, 16, 64, 8, 2, 4096
_T = _B * _S
_CAP = int(_B * _S * 1.25 * _K / _E)
_NSLOT = _E * _CAP
_DROWS = _NSLOT + 256
_TEMP = 0.1
_BF = jnp.bfloat16
_F32 = jnp.float32


def _qkvg_body(x_ref, sc_ref, bi_ref, w_ref, o_ref):
    x = x_ref[...]
    m = jnp.mean(x, axis=1, keepdims=True)
    v = jnp.mean((x - m) ** 2, axis=1, keepdims=True)
    h = (x - m) * jax.lax.rsqrt(v + 1e-5) * sc_ref[...] + bi_ref[...]
    o_ref[...] = jnp.dot(h.astype(_BF), w_ref[...],
                         preferred_element_type=_F32).astype(_BF)


def _qkvg_call(x, scale, bias, wcat):
    return pl.pallas_call(
        _qkvg_body,
        grid=(8, 8),
        in_specs=[
            pl.BlockSpec((512, _D), lambda i, j: (i, 0)),
            pl.BlockSpec((1, _D), lambda i, j: (0, 0)),
            pl.BlockSpec((1, _D), lambda i, j: (0, 0)),
            pl.BlockSpec((_D, 512), lambda i, j: (0, j)),
        ],
        out_specs=pl.BlockSpec((512, 512), lambda i, j: (i, j)),
        out_shape=jax.ShapeDtypeStruct((_T, 4 * _D), _BF),
        compiler_params=pltpu.CompilerParams(
            dimension_semantics=("parallel", "parallel")),
    )(x, scale, bias, wcat)


_BQ = 256


def _flash_body(q_ref, k_ref, v_ref, cs_ref, sn_ref, o_ref):
    i = pl.program_id(1)
    r64 = jax.lax.broadcasted_iota(jnp.int32, (_HD, _HD), 0)
    c64 = jax.lax.broadcasted_iota(jnp.int32, (_HD, _HD), 1)
    P = (r64 == (c64 + _HD // 2) % _HD).astype(_BF)
    q = q_ref[...].reshape(_BQ, _HD)
    qs = jnp.dot(q, P, preferred_element_type=_F32)
    cos_q = cs_ref[pl.ds(i * _BQ, _BQ), :]
    sin_q = sn_ref[pl.ds(i * _BQ, _BQ), :]
    qr = ((q.astype(_F32) * cos_q + qs * sin_q) * 0.125).astype(_BF)
    k = k_ref[...].reshape(_S, _HD)
    ks = jnp.dot(k, P, preferred_element_type=_F32)
    kr = (k.astype(_F32) * cs_ref[...] + ks * sn_ref[...]).astype(_BF)
    s = jax.lax.dot_general(qr, kr, (((1,), (1,)), ((), ())),
                            preferred_element_type=_F32)
    qpos = i * _BQ + jax.lax.broadcasted_iota(jnp.int32, (_BQ, _S), 0)
    kpos = jax.lax.broadcasted_iota(jnp.int32, (_BQ, _S), 1)
    s = jnp.where(qpos >= kpos, s, -1e9)
    m = jnp.max(s, axis=1, keepdims=True)
    p = jnp.exp(s - m)
    l = jnp.sum(p, axis=1, keepdims=True)
    v = v_ref[...].reshape(_S, _HD)
    o = jnp.dot(p.astype(_BF), v, preferred_element_type=_F32) / l
    o_ref[...] = o.astype(_BF).reshape(_BQ, 1, 1, _HD)


def _flash_call(qkvg, cos, sin):
    nq = _S // _BQ
    qkvg4 = qkvg.reshape(_T, 64, 1, _HD)
    attn4 = pl.pallas_call(
        _flash_body,
        grid=(_B * _H, nq),
        in_specs=[
            pl.BlockSpec((_BQ, 1, 1, _HD),
                         lambda bh, i: (bh // _H * nq + i, bh % _H, 0, 0)),
            pl.BlockSpec((_S, 1, 1, _HD),
                         lambda bh, i: (bh // _H, _H + bh % _H, 0, 0)),
            pl.BlockSpec((_S, 1, 1, _HD),
                         lambda bh, i: (bh // _H, 2 * _H + bh % _H, 0, 0)),
            pl.BlockSpec((_S, _HD), lambda bh, i: (0, 0)),
            pl.BlockSpec((_S, _HD), lambda bh, i: (0, 0)),
        ],
        out_specs=pl.BlockSpec((_BQ, 1, 1, _HD),
                               lambda bh, i: (bh // _H * nq + i, bh % _H, 0, 0)),
        out_shape=jax.ShapeDtypeStruct((_T, _H, 1, _HD), _BF),
        compiler_params=pltpu.CompilerParams(
            dimension_semantics=("parallel", "parallel")),
    )(qkvg4, qkvg4, qkvg4, cos, sin)
    return attn4.reshape(_T, _H * _HD)


def _wo_body(a_ref, wo_ref, g_ref, bg_ref, ls_ref, x_ref, o_ref):
    a = jnp.dot(a_ref[...], wo_ref[...], preferred_element_type=_F32)
    gate = jax.nn.sigmoid(g_ref[...].astype(_F32) + bg_ref[...])
    o_ref[...] = x_ref[...] + ls_ref[...] * (gate * a)


def _wo_call(attn, wo, qkvg, bg, ls1, x):
    return pl.pallas_call(
        _wo_body,
        grid=(8, 4),
        in_specs=[
            pl.BlockSpec((512, _D), lambda i, j: (i, 0)),
            pl.BlockSpec((_D, 256), lambda i, j: (0, j)),
            pl.BlockSpec((512, 256), lambda i, j: (i, 12 + j)),
            pl.BlockSpec((1, 256), lambda i, j: (0, j)),
            pl.BlockSpec((1, 256), lambda i, j: (0, j)),
            pl.BlockSpec((512, 256), lambda i, j: (i, j)),
        ],
        out_specs=pl.BlockSpec((512, 256), lambda i, j: (i, j)),
        out_shape=jax.ShapeDtypeStruct((_T, _D), _F32),
        compiler_params=pltpu.CompilerParams(
            dimension_semantics=("parallel", "parallel")),
    )(attn, wo, qkvg, bg, ls1, x)


def _ln2_body(x_ref, sc_ref, bi_ref, wr_ref, h2_ref, tw_ref, ti_ref):
    x = x_ref[...]
    m = jnp.mean(x, axis=1, keepdims=True)
    v = jnp.mean((x - m) ** 2, axis=1, keepdims=True)
    h = (x - m) * jax.lax.rsqrt(v + 1e-5) * sc_ref[...] + bi_ref[...]
    h2_ref[...] = h
    logits = jnp.dot(h, wr_ref[...], preferred_element_type=_F32) / _TEMP
    lane = jax.lax.broadcasted_iota(jnp.int32, logits.shape, 1)
    valid = lane < _E
    mx = jnp.max(jnp.where(valid, logits, -1e30), axis=1, keepdims=True)
    p = jnp.where(valid, jnp.exp(logits - mx), 0.0)
    probs = p / jnp.sum(p, axis=1, keepdims=True)
    v1 = jnp.max(probs, axis=1, keepdims=True)
    i1 = jnp.min(jnp.where(probs == v1, lane, 128), axis=1, keepdims=True)
    p2 = jnp.where(lane == i1, -1.0, probs)
    v2 = jnp.max(p2, axis=1, keepdims=True)
    i2 = jnp.min(jnp.where(p2 == v2, lane, 128), axis=1, keepdims=True)
    tot = v1 + v2
    k8 = jax.lax.broadcasted_iota(jnp.int32, (x.shape[0], 8), 1)
    tw_ref[...] = jnp.where(k8 == 0, v1 / tot, jnp.where(k8 == 1, v2 / tot, 0.0))
    ti_ref[...] = jnp.where(k8 == 0, i1, jnp.where(k8 == 1, i2, 0))


def _ln2_call(x2, scale, bias, wr_pad):
    return pl.pallas_call(
        _ln2_body,
        grid=(16,),
        in_specs=[
            pl.BlockSpec((256, _D), lambda i: (i, 0)),
            pl.BlockSpec((1, _D), lambda i: (0, 0)),
            pl.BlockSpec((1, _D), lambda i: (0, 0)),
            pl.BlockSpec((_D, 128), lambda i: (0, 0)),
        ],
        out_specs=[
            pl.BlockSpec((256, _D), lambda i: (i, 0)),
            pl.BlockSpec((256, 8), lambda i: (i, 0)),
            pl.BlockSpec((256, 8), lambda i: (i, 0)),
        ],
        out_shape=[
            jax.ShapeDtypeStruct((_T, _D), _F32),
            jax.ShapeDtypeStruct((_T, 8), _F32),
            jax.ShapeDtypeStruct((_T, 8), jnp.int32),
        ],
        compiler_params=pltpu.CompilerParams(
            dimension_semantics=("parallel",)),
    )(x2, scale, bias, wr_pad)


_PC = 1024


def _pos_body(fe_ref, ss_ref, sg_ref, kp_ref):
    r = jax.lax.broadcasted_iota(jnp.int32, (_PC, _PC), 0)
    c = jax.lax.broadcasted_iota(jnp.int32, (_PC, _PC), 1)
    tri = (r >= c).astype(_F32)
    lane = jax.lax.broadcasted_iota(jnp.int32, (_PC, 128), 1)

    q4 = jax.lax.broadcasted_iota(jnp.int32, (_PC, 4), 1)

    def chunk(ci, carry):
        e = fe_ref[pl.ds(ci * _PC, _PC), :]
        ohc = (e == lane).astype(_F32)
        cum = jnp.dot(tri, ohc, preferred_element_type=_F32) + carry
        pos = jnp.sum(ohc * cum, axis=1, keepdims=True).astype(jnp.int32) - 1
        keep = pos < _CAP
        slot = e * _CAP + jnp.minimum(pos, _CAP - 1)
        ss_ref[pl.ds(ci * _PC, _PC), :] = jnp.where(keep, slot, _NSLOT) * 4 + q4
        sg_ref[pl.ds(ci * _PC, _PC), :] = jnp.where(keep, slot, 0) * 4 + q4
        kp_ref[pl.ds(ci * _PC, _PC), :] = keep.astype(_F32)
        return carry + jnp.sum(ohc, axis=0, keepdims=True)

    jax.lax.fori_loop(0, (2 * _T) // _PC, chunk, jnp.zeros((1, 128), _F32))


def _pos_call(fe):
    return pl.pallas_call(
        _pos_body,
        in_specs=[pl.BlockSpec((2 * _T, 1), lambda: (0, 0))],
        out_specs=[
            pl.BlockSpec((2 * _T, 4), lambda: (0, 0)),
            pl.BlockSpec((2 * _T, 4), lambda: (0, 0)),
            pl.BlockSpec((2 * _T, 1), lambda: (0, 0)),
        ],
        out_shape=[
            jax.ShapeDtypeStruct((2 * _T, 4), jnp.int32),
            jax.ShapeDtypeStruct((2 * _T, 4), jnp.int32),
            jax.ShapeDtypeStruct((2 * _T, 1), _F32),
        ],
    )(fe)


def _sc_mesh():
    return plsc.VectorSubcoreMesh(core_axis_name="core",
                                  subcore_axis_name="subcore")


def _dispatch_scatter(h2q, slots4):
    nw = (4 * _T) // 128

    @pl.kernel(out_type=jax.ShapeDtypeStruct((4 * _DROWS, _D // 4), _F32),
               mesh=_sc_mesh(), scratch_types=[])
    def k(x_hbm, i_hbm, o_hbm):
        def body(x_vmem, i_vmem):
            pltpu.sync_copy(x_vmem, o_hbm.at[i_vmem.at[0]])

        pltpu.emit_pipeline(
            body,
            grid=(_K, nw),
            in_specs=[
                pl.BlockSpec((128, _D // 4), index_map=lambda kk, i: (i, 0)),
                pl.BlockSpec((1, 128), index_map=lambda kk, i: (0, kk * nw + i)),
            ],
            out_specs=[],
            core_axis_name=("core", "subcore"),
            dimension_semantics=(pltpu.PARALLEL, pltpu.PARALLEL),
        )(x_hbm, i_hbm)

    return k(h2q, slots4)


def _combine_gather(eoutq, slots4):
    nw = (2 * 4 * _T) // 128

    @pl.kernel(out_type=jax.ShapeDtypeStruct((2 * 4 * _T, _D // 4), _F32),
               mesh=_sc_mesh(), scratch_types=[])
    def k(x_hbm, i_hbm, o_hbm):
        def body(i_vmem, o_vmem):
            pltpu.sync_copy(x_hbm.at[i_vmem.at[0]], o_vmem)

        pltpu.emit_pipeline(
            body,
            grid=(nw,),
            in_specs=[pl.BlockSpec((1, 128), index_map=lambda i: (0, i))],
            out_specs=[pl.BlockSpec((128, _D // 4), index_map=lambda i: (i, 0))],
            core_axis_name=("core", "subcore"),
            dimension_semantics=(pltpu.PARALLEL,),
        )(i_hbm, o_hbm)

    return k(eoutq, slots4)


def _ffn_body(d_ref, w1_ref, b1_ref, w2_ref, b2_ref, o_ref):
    h = jnp.dot(d_ref[...].astype(_BF), w1_ref[0],
                preferred_element_type=_F32) + b1_ref[0]
    h = jnp.maximum(h, 0.0).astype(_BF)
    o_ref[...] = jnp.dot(h, w2_ref[0], preferred_element_type=_F32) + b2_ref[0]


def _ffn_call(disp, w1, b1, w2, b2):
    nc = _CAP // 256
    return pl.pallas_call(
        _ffn_body,
        grid=(_E, nc),
        in_specs=[
            pl.BlockSpec((256, _D), lambda e, c: (e * nc + c, 0)),
            pl.BlockSpec((1, _D, _FF), lambda e, c: (e, 0, 0)),
            pl.BlockSpec((1, 1, _FF), lambda e, c: (e, 0, 0)),
            pl.BlockSpec((1, _FF, _D), lambda e, c: (e, 0, 0)),
            pl.BlockSpec((1, 1, _D), lambda e, c: (e, 0, 0)),
        ],
        out_specs=pl.BlockSpec((256, _D), lambda e, c: (e * nc + c, 0)),
        out_shape=jax.ShapeDtypeStruct((_NSLOT, _D), _F32),
        compiler_params=pltpu.CompilerParams(
            dimension_semantics=("parallel", "parallel")),
    )(disp, w1, b1, w2, b2)


def _comb_body(g0_ref, g1_ref, tw_ref, k0_ref, k1_ref, x_ref, ls_ref, o_ref):
    w0 = tw_ref[...][:, 0:1] * k0_ref[...]
    w1 = tw_ref[...][:, 1:2] * k1_ref[...]
    moe = w0 * g0_ref[...] + w1 * g1_ref[...]
    o_ref[...] = x_ref[...] + ls_ref[...] * moe


def _comb_call(g0, g1, tw, k0, k1, x2, ls2):
    return pl.pallas_call(
        _comb_body,
        grid=(8, 4),
        in_specs=[
            pl.BlockSpec((512, 256), lambda i, j: (i, j)),
            pl.BlockSpec((512, 256), lambda i, j: (i, j)),
            pl.BlockSpec((512, 8), lambda i, j: (i, 0)),
            pl.BlockSpec((512, 1), lambda i, j: (i, 0)),
            pl.BlockSpec((512, 1), lambda i, j: (i, 0)),
            pl.BlockSpec((512, 256), lambda i, j: (i, j)),
            pl.BlockSpec((1, 256), lambda i, j: (0, j)),
        ],
        out_specs=pl.BlockSpec((512, 256), lambda i, j: (i, j)),
        out_shape=jax.ShapeDtypeStruct((_T, _D), _F32),
        compiler_params=pltpu.CompilerParams(
            dimension_semantics=("parallel", "parallel")),
    )(g0, g1, tw, k0, k1, x2, ls2)


def kernel(hidden_states, ln1_scale, ln1_bias, Wq, Wk, Wv, Wo, Wg, bg, ls1,
           ln2_scale, ln2_bias, Wr, w1, b1, w2, b2, ls2):
    x = hidden_states.reshape(_T, _D)
    wcat = jnp.concatenate([Wq, Wk, Wv, Wg], axis=1).astype(_BF)
    qkvg = _qkvg_call(x, ln1_scale.reshape(1, _D), ln1_bias.reshape(1, _D), wcat)

    half = _HD // 2
    posn = jnp.arange(_S, dtype=_F32)[:, None]
    freqs = 1.0 / (10000.0 ** (jnp.arange(half, dtype=_F32) / half))
    ang = posn * freqs
    cos64 = jnp.concatenate([jnp.cos(ang), jnp.cos(ang)], axis=1)
    ssin64 = jnp.concatenate([-jnp.sin(ang), jnp.sin(ang)], axis=1)

    attn = _flash_call(qkvg, cos64, ssin64)
    x2 = _wo_call(attn, Wo.astype(_BF), qkvg, bg.reshape(1, _D),
                  ls1.reshape(1, _D), x)

    wr_pad = jnp.pad(Wr, ((0, 0), (0, 128 - _E)))
    h2b, tw, ti = _ln2_call(x2, ln2_scale.reshape(1, _D),
                            ln2_bias.reshape(1, _D), wr_pad)

    fe = jnp.concatenate([ti[:, 0], ti[:, 1]]).reshape(2 * _T, 1)
    ss, sg, kp = _pos_call(fe)

    disp4 = _dispatch_scatter(h2b.reshape(4 * _T, _D // 4),
                              ss.reshape(1, 4 * 2 * _T))
    eout = _ffn_call(disp4.reshape(_DROWS, _D), w1.astype(_BF),
                     b1.reshape(_E, 1, _FF), w2.astype(_BF),
                     b2.reshape(_E, 1, _D))
    gathered = _combine_gather(eout.reshape(4 * _NSLOT, _D // 4),
                               sg.reshape(1, 4 * 2 * _T)).reshape(2 * _T, _D)

    xo = _comb_call(gathered[:_T], gathered[_T:], tw, kp[:_T], kp[_T:],
                    x2, ls2.reshape(1, _D))
    return xo.reshape(_B, _S, _D)
